# baseline (speedup 1.0000x reference)
import numpy as np
import jax
import jax.numpy as jnp
from jax.experimental import pallas as pl
from jax.experimental.pallas import tpu as pltpu

_CHANNELS = 256
_SPATIAL = 32 * 32 * 32

_rng = np.random.default_rng(0)
_M_RATIO = 0.85 + _rng.random() * 0.05
_M = int(np.ceil(_CHANNELS * _M_RATIO))
_RNG_DROP = _rng.normal(loc=0.2, scale=0.05)
if _RNG_DROP < 0:
    _RNG_DROP = 0.0
_RNG_DROP = np.float32(_RNG_DROP)

_C_BLK = 32


def _sum_kernel(x_ref, out_ref):
    s = jnp.sum(x_ref[...], axis=-1)
    out_ref[...] = s[:, None, None, :]


def _mask_kernel(ssum_ref, gumbel_ref, ru_ref, out_ref):
    scores = ssum_ref[...] * np.float32(1.0 / _SPATIAL)
    p = jnp.log(jnp.maximum(scores, 1e-30)) + gumbel_ref[...]
    bs = p.shape[0]
    pi = p[:, :, None]
    pj = p[:, None, :]
    ji = jax.lax.broadcasted_iota(jnp.int32, (bs, _CHANNELS, _CHANNELS), 1)
    jj = jax.lax.broadcasted_iota(jnp.int32, (bs, _CHANNELS, _CHANNELS), 2)
    beats = (pj > pi) | ((pj == pi) & (jj < ji))
    rank = jnp.sum(beats.astype(jnp.float32), axis=2)
    sel = rank < np.float32(_M)
    keep = ru_ref[...] > _RNG_DROP
    out_ref[...] = (sel & keep).astype(jnp.float32)


def _mul_kernel(x_ref, m_ref, out_ref):
    out_ref[...] = x_ref[...] * m_ref[...]


def kernel(inputs):
    bs, chns = inputs.shape[0], inputs.shape[1]
    x = inputs.reshape(bs, chns, -1)

    key = jax.random.key(42)
    k1, k2 = jax.random.split(key, 2)
    u = jax.random.uniform(k1, (bs, chns), minval=1e-10, maxval=1.0)
    gumbel = -jnp.log(-jnp.log(u))
    ru = jax.random.uniform(k2, (bs, chns))

    n_cblk = chns // _C_BLK

    ssum = pl.pallas_call(
        _sum_kernel,
        grid=(bs, n_cblk),
        in_specs=[pl.BlockSpec((1, _C_BLK, _SPATIAL), lambda b, c: (b, c, 0))],
        out_specs=pl.BlockSpec((1, 1, 1, _C_BLK), lambda b, c: (b, c, 0, 0)),
        out_shape=jax.ShapeDtypeStruct((bs, n_cblk, 1, _C_BLK), jnp.float32),
        compiler_params=pltpu.CompilerParams(
            dimension_semantics=("parallel", "parallel")
        ),
    )(x)
    ssum = ssum.reshape(bs, chns)

    dmask = pl.pallas_call(
        _mask_kernel,
        out_shape=jax.ShapeDtypeStruct((bs, chns), jnp.float32),
    )(ssum, gumbel, ru)

    out = pl.pallas_call(
        _mul_kernel,
        grid=(bs, n_cblk),
        in_specs=[
            pl.BlockSpec((1, _C_BLK, _SPATIAL), lambda b, c: (b, c, 0)),
            pl.BlockSpec((1, _C_BLK, 1), lambda b, c: (b, c, 0)),
        ],
        out_specs=pl.BlockSpec((1, _C_BLK, _SPATIAL), lambda b, c: (b, c, 0)),
        out_shape=jax.ShapeDtypeStruct((bs, chns, _SPATIAL), jnp.float32),
        compiler_params=pltpu.CompilerParams(
            dimension_semantics=("parallel", "parallel")
        ),
    )(x, dmask[:, :, None])

    return out.reshape(inputs.shape)


# speedup vs baseline: 2.6890x; 2.6890x over previous
import numpy as np
import jax
import jax.numpy as jnp
from jax.experimental import pallas as pl
from jax.experimental.pallas import tpu as pltpu

_CHANNELS = 256
_SPATIAL = 32 * 32 * 32
_S_BLK = 4096

_rng = np.random.default_rng(0)
_M_RATIO = 0.85 + _rng.random() * 0.05
_M = int(np.ceil(_CHANNELS * _M_RATIO))
_RNG_DROP = _rng.normal(loc=0.2, scale=0.05)
if _RNG_DROP < 0:
    _RNG_DROP = 0.0
_RNG_DROP = np.float32(_RNG_DROP)


def _sum_kernel(x_ref, out_ref):
    s = pl.program_id(1)

    @pl.when(s == 0)
    def _init():
        out_ref[...] = jnp.zeros_like(out_ref)

    out_ref[...] += jnp.sum(x_ref[...], axis=1)[:, None, :]


def _mask_kernel(ssum_ref, gumbel_ref, ru_ref, out_ref):
    scores = ssum_ref[...] * np.float32(1.0 / _SPATIAL)
    p = jnp.log(jnp.maximum(scores, 1e-30)) + gumbel_ref[...]
    p = p[:, 0, :]
    bs = p.shape[0]
    pi = p[:, :, None]
    pj = p[:, None, :]
    ii = jax.lax.broadcasted_iota(jnp.int32, (bs, _CHANNELS, _CHANNELS), 1)
    jj = jax.lax.broadcasted_iota(jnp.int32, (bs, _CHANNELS, _CHANNELS), 2)
    beats = (pj > pi) | ((pj == pi) & (jj < ii))
    rank = jnp.sum(beats.astype(jnp.float32), axis=2)
    sel = rank < np.float32(_M)
    keep = ru_ref[...][:, 0, :] > _RNG_DROP
    out_ref[...] = (sel & keep).astype(jnp.float32)[:, None, :]


def _mul_kernel(x_ref, m_ref, out_ref):
    out_ref[...] = x_ref[...] * m_ref[...]


def kernel(inputs):
    bs, chns = inputs.shape[0], inputs.shape[1]
    x = inputs.transpose(0, 2, 3, 4, 1).reshape(bs, _SPATIAL, chns)

    key = jax.random.key(42)
    k1, k2 = jax.random.split(key, 2)
    u = jax.random.uniform(k1, (bs, chns), minval=1e-10, maxval=1.0)
    gumbel = (-jnp.log(-jnp.log(u)))[:, None, :]
    ru = jax.random.uniform(k2, (bs, chns))[:, None, :]

    n_sblk = _SPATIAL // _S_BLK

    ssum = pl.pallas_call(
        _sum_kernel,
        grid=(bs, n_sblk),
        in_specs=[pl.BlockSpec((1, _S_BLK, chns), lambda b, s: (b, s, 0))],
        out_specs=pl.BlockSpec((1, 1, chns), lambda b, s: (b, 0, 0)),
        out_shape=jax.ShapeDtypeStruct((bs, 1, chns), jnp.float32),
        compiler_params=pltpu.CompilerParams(
            dimension_semantics=("parallel", "arbitrary")
        ),
    )(x)

    dmask = pl.pallas_call(
        _mask_kernel,
        out_shape=jax.ShapeDtypeStruct((bs, 1, chns), jnp.float32),
    )(ssum, gumbel, ru)

    out = pl.pallas_call(
        _mul_kernel,
        grid=(bs, n_sblk),
        in_specs=[
            pl.BlockSpec((1, _S_BLK, chns), lambda b, s: (b, s, 0)),
            pl.BlockSpec((1, 1, chns), lambda b, s: (b, 0, 0)),
        ],
        out_specs=pl.BlockSpec((1, _S_BLK, chns), lambda b, s: (b, s, 0)),
        out_shape=jax.ShapeDtypeStruct((bs, _SPATIAL, chns), jnp.float32),
        compiler_params=pltpu.CompilerParams(
            dimension_semantics=("parallel", "parallel")
        ),
    )(x, dmask)

    return out.reshape(bs, 32, 32, 32, chns).transpose(0, 4, 1, 2, 3)


# speedup vs baseline: 3.6826x; 1.3695x over previous
import numpy as np
import jax
import jax.numpy as jnp
from jax.experimental import pallas as pl
from jax.experimental.pallas import tpu as pltpu

_CHANNELS = 256
_SPATIAL = 32 * 32 * 32
_S_BLK = 4096
_N_SBLK = _SPATIAL // _S_BLK

_rng = np.random.default_rng(0)
_M_RATIO = 0.85 + _rng.random() * 0.05
_M = int(np.ceil(_CHANNELS * _M_RATIO))
_RNG_DROP = _rng.normal(loc=0.2, scale=0.05)
if _RNG_DROP < 0:
    _RNG_DROP = 0.0
_RNG_DROP = np.float32(_RNG_DROP)


def _fused_kernel(x_ref, gumbel_ref, ru_ref, out_ref, data_ref, sums_ref, mask_ref):
    ph = pl.program_id(1)
    s = pl.program_id(2)

    @pl.when(ph == 0)
    def _load():
        v = x_ref[...]
        data_ref[pl.ds(s, 1)] = v
        part = jnp.sum(v, axis=1)

        @pl.when(s == 0)
        def _init():
            sums_ref[...] = part

        @pl.when(s != 0)
        def _acc():
            sums_ref[...] += part

    @pl.when((ph == 1) & (s == 0))
    def _mask():
        scores = sums_ref[...] * np.float32(1.0 / _SPATIAL)
        p = jnp.log(jnp.maximum(scores, 1e-30)) + gumbel_ref[...][:, 0, :]
        pi = p[:, :, None]
        pj = p[:, None, :]
        ii = jax.lax.broadcasted_iota(jnp.int32, (1, _CHANNELS, _CHANNELS), 1)
        jj = jax.lax.broadcasted_iota(jnp.int32, (1, _CHANNELS, _CHANNELS), 2)
        beats = (pj > pi) | ((pj == pi) & (jj < ii))
        rank = jnp.sum(beats.astype(jnp.float32), axis=2)
        sel = rank < np.float32(_M)
        keep = ru_ref[...][:, 0, :] > _RNG_DROP
        mask_ref[...] = (sel & keep).astype(jnp.float32)

    @pl.when(ph == 1)
    def _mul():
        out_ref[...] = data_ref[pl.ds(s, 1)] * mask_ref[...][:, None, :]


def kernel(inputs):
    bs, chns = inputs.shape[0], inputs.shape[1]
    x = inputs.transpose(0, 2, 3, 4, 1).reshape(bs, _SPATIAL, chns)

    key = jax.random.key(42)
    k1, k2 = jax.random.split(key, 2)
    u = jax.random.uniform(k1, (bs, chns), minval=1e-10, maxval=1.0)
    gumbel = (-jnp.log(-jnp.log(u)))[:, None, :]
    ru = jax.random.uniform(k2, (bs, chns))[:, None, :]

    out = pl.pallas_call(
        _fused_kernel,
        grid=(bs, 2, _N_SBLK),
        in_specs=[
            pl.BlockSpec(
                (1, _S_BLK, chns),
                lambda b, ph, s: (b, s * (1 - ph) + (_N_SBLK - 1) * ph, 0),
            ),
            pl.BlockSpec((1, 1, chns), lambda b, ph, s: (b, 0, 0)),
            pl.BlockSpec((1, 1, chns), lambda b, ph, s: (b, 0, 0)),
        ],
        out_specs=pl.BlockSpec((1, _S_BLK, chns), lambda b, ph, s: (b, s * ph, 0)),
        out_shape=jax.ShapeDtypeStruct((bs, _SPATIAL, chns), jnp.float32),
        scratch_shapes=[
            pltpu.VMEM((_N_SBLK, _S_BLK, chns), jnp.float32),
            pltpu.VMEM((1, chns), jnp.float32),
            pltpu.VMEM((1, chns), jnp.float32),
        ],
        compiler_params=pltpu.CompilerParams(
            dimension_semantics=("arbitrary", "arbitrary", "arbitrary")
        ),
    )(x, gumbel, ru)

    return out.reshape(bs, 32, 32, 32, chns).transpose(0, 4, 1, 2, 3)


# speedup vs baseline: 3.6927x; 1.0028x over previous
import numpy as np
import jax
import jax.numpy as jnp
from jax.experimental import pallas as pl
from jax.experimental.pallas import tpu as pltpu

_CHANNELS = 256
_SPATIAL = 32 * 32 * 32
_S_BLK = 4096
_N_SBLK = _SPATIAL // _S_BLK

_rng = np.random.default_rng(0)
_M_RATIO = 0.85 + _rng.random() * 0.05
_M = int(np.ceil(_CHANNELS * _M_RATIO))
_RNG_DROP = _rng.normal(loc=0.2, scale=0.05)
if _RNG_DROP < 0:
    _RNG_DROP = 0.0
_RNG_DROP = np.float32(_RNG_DROP)


def _fused_kernel(x_ref, gumbel_ref, ru_ref, out_ref, data_ref, sums_ref, mask_ref):
    ph = pl.program_id(1)
    s = pl.program_id(2)

    @pl.when(ph == 0)
    def _load():
        v = x_ref[...]
        data_ref[pl.ds(s, 1)] = v
        part = jnp.sum(v, axis=1)

        @pl.when(s == 0)
        def _init():
            sums_ref[...] = part

        @pl.when(s != 0)
        def _acc():
            sums_ref[...] += part

    @pl.when((ph == 1) & (s == 0))
    def _mask():
        scores = sums_ref[...] * np.float32(1.0 / _SPATIAL)
        p = jnp.log(jnp.maximum(scores, 1e-30)) + gumbel_ref[...][:, 0, :]
        pi = p[:, :, None]
        pj = p[:, None, :]
        ii = jax.lax.broadcasted_iota(jnp.int32, (1, _CHANNELS, _CHANNELS), 1)
        jj = jax.lax.broadcasted_iota(jnp.int32, (1, _CHANNELS, _CHANNELS), 2)
        beats = (pj > pi) | ((pj == pi) & (jj < ii))
        rank = jnp.sum(beats.astype(jnp.float32), axis=2)
        sel = rank < np.float32(_M)
        keep = ru_ref[...][:, 0, :] > _RNG_DROP
        mask_ref[...] = (sel & keep).astype(jnp.float32)

    @pl.when(ph == 1)
    def _mul():
        out_ref[...] = data_ref[pl.ds(s, 1)] * mask_ref[...][:, None, :]


def kernel(inputs):
    bs, chns = inputs.shape[0], inputs.shape[1]
    x = inputs.transpose(0, 2, 3, 4, 1).reshape(bs, _SPATIAL, chns)

    key = jax.random.key(42)
    k1, k2 = jax.random.split(key, 2)
    u = jax.random.uniform(k1, (bs, chns), minval=1e-10, maxval=1.0)
    gumbel = (-jnp.log(-jnp.log(u)))[:, None, :]
    ru = jax.random.uniform(k2, (bs, chns))[:, None, :]

    out = pl.pallas_call(
        _fused_kernel,
        grid=(bs, 2, _N_SBLK),
        in_specs=[
            pl.BlockSpec(
                (1, _S_BLK, chns),
                lambda b, ph, s: (b, s * (1 - ph) + (_N_SBLK - 1) * ph, 0),
            ),
            pl.BlockSpec((1, 1, chns), lambda b, ph, s: (b, 0, 0)),
            pl.BlockSpec((1, 1, chns), lambda b, ph, s: (b, 0, 0)),
        ],
        out_specs=pl.BlockSpec((1, _S_BLK, chns), lambda b, ph, s: (b, s * ph, 0)),
        out_shape=jax.ShapeDtypeStruct((bs, _SPATIAL, chns), jnp.float32),
        scratch_shapes=[
            pltpu.VMEM((_N_SBLK, _S_BLK, chns), jnp.float32),
            pltpu.VMEM((1, chns), jnp.float32),
            pltpu.VMEM((1, chns), jnp.float32),
        ],
        compiler_params=pltpu.CompilerParams(
            dimension_semantics=("parallel", "arbitrary", "arbitrary")
        ),
    )(x, gumbel, ru)

    return out.reshape(bs, 32, 32, 32, chns).transpose(0, 4, 1, 2, 3)


# speedup vs baseline: 4.0691x; 1.1019x over previous
import numpy as np
import jax
import jax.numpy as jnp
from jax.experimental import pallas as pl
from jax.experimental.pallas import tpu as pltpu

_CHANNELS = 256
_SPATIAL = 32 * 32 * 32
_S_BLK = 4096
_N_SBLK = _SPATIAL // _S_BLK
_RING = _N_SBLK + 1
_BS = 8

_rng = np.random.default_rng(0)
_M_RATIO = 0.85 + _rng.random() * 0.05
_M = int(np.ceil(_CHANNELS * _M_RATIO))
_RNG_DROP = _rng.normal(loc=0.2, scale=0.05)
if _RNG_DROP < 0:
    _RNG_DROP = 0.0
_RNG_DROP = np.float32(_RNG_DROP)


def _fused_kernel(x_ref, gumbel_ref, ru_ref, out_ref, data_ref, sums_ref, mask_ref):
    vb = pl.program_id(0)
    s = pl.program_id(1)
    slot = (vb * _N_SBLK + s) % _RING

    @pl.when((vb >= 1) & (s == 0))
    def _mask():
        ssum = sums_ref[pl.ds((vb + 1) % 2, 1)]
        scores = ssum * np.float32(1.0 / _SPATIAL)
        p = jnp.log(jnp.maximum(scores, 1e-30)) + gumbel_ref[...][:, 0, :]
        pi = p[:, :, None]
        pj = p[:, None, :]
        ii = jax.lax.broadcasted_iota(jnp.int32, (1, _CHANNELS, _CHANNELS), 1)
        jj = jax.lax.broadcasted_iota(jnp.int32, (1, _CHANNELS, _CHANNELS), 2)
        beats = (pj > pi) | ((pj == pi) & (jj < ii))
        rank = jnp.sum(beats.astype(jnp.float32), axis=2)
        sel = rank < np.float32(_M)
        keep = ru_ref[...][:, 0, :] > _RNG_DROP
        mask_ref[...] = (sel & keep).astype(jnp.float32)

    @pl.when(vb < _BS)
    def _load():
        v = x_ref[...]
        data_ref[pl.ds(slot, 1)] = v
        part = jnp.sum(v, axis=1)

        @pl.when(s == 0)
        def _init():
            sums_ref[pl.ds(vb % 2, 1)] = part

        @pl.when(s != 0)
        def _acc():
            sums_ref[pl.ds(vb % 2, 1)] += part

    @pl.when(vb >= 1)
    def _mul():
        prev = data_ref[pl.ds((slot + 1) % _RING, 1)]
        out_ref[...] = prev * mask_ref[...][:, None, :]


def kernel(inputs):
    bs, chns = inputs.shape[0], inputs.shape[1]
    x = inputs.transpose(0, 2, 3, 4, 1).reshape(bs, _SPATIAL, chns)

    key = jax.random.key(42)
    k1, k2 = jax.random.split(key, 2)
    u = jax.random.uniform(k1, (bs, chns), minval=1e-10, maxval=1.0)
    gumbel = (-jnp.log(-jnp.log(u)))[:, None, :]
    ru = jax.random.uniform(k2, (bs, chns))[:, None, :]

    out = pl.pallas_call(
        _fused_kernel,
        grid=(bs + 1, _N_SBLK),
        in_specs=[
            pl.BlockSpec(
                (1, _S_BLK, chns),
                lambda vb, s: (
                    jnp.minimum(vb, _BS - 1),
                    jnp.maximum(s, (_N_SBLK - 1) * (vb // _BS)),
                    0,
                ),
            ),
            pl.BlockSpec((1, 1, chns), lambda vb, s: (jnp.maximum(vb - 1, 0), 0, 0)),
            pl.BlockSpec((1, 1, chns), lambda vb, s: (jnp.maximum(vb - 1, 0), 0, 0)),
        ],
        out_specs=pl.BlockSpec(
            (1, _S_BLK, chns),
            lambda vb, s: (jnp.maximum(vb - 1, 0), s * jnp.minimum(vb, 1), 0),
        ),
        out_shape=jax.ShapeDtypeStruct((bs, _SPATIAL, chns), jnp.float32),
        scratch_shapes=[
            pltpu.VMEM((_RING, _S_BLK, chns), jnp.float32),
            pltpu.VMEM((2, chns), jnp.float32),
            pltpu.VMEM((1, chns), jnp.float32),
        ],
        compiler_params=pltpu.CompilerParams(
            dimension_semantics=("arbitrary", "arbitrary")
        ),
    )(x, gumbel, ru)

    return out.reshape(bs, 32, 32, 32, chns).transpose(0, 4, 1, 2, 3)
